# baseline (speedup 1.0000x reference)
import functools

import jax
import jax.numpy as jnp
from jax import lax
from jax.experimental import pallas as pl
from jax.experimental.pallas import tpu as pltpu
from jax.experimental.pallas import tpu_sc as plsc

N_NODES_PAD = 10240
NC = 2
NS = 16
NW = NC * NS
CHUNK = 128
ROWS_PER_SUB = N_NODES_PAD // NS

_mesh = plsc.VectorSubcoreMesh(core_axis_name="c", subcore_axis_name="s")


DEG_ROWS = N_NODES_PAD // CHUNK


def _sc_degree(dst_slab, zeros_deg, k_chunks):
    @functools.partial(
        pl.kernel,
        out_type=jax.ShapeDtypeStruct((NC, DEG_ROWS, CHUNK), jnp.float32),
        mesh=_mesh,
        compiler_params=pltpu.CompilerParams(needs_layout_passes=False),
        scratch_types=[
            pltpu.VMEM((k_chunks, CHUNK), jnp.int32),
            pltpu.VMEM((DEG_ROWS, CHUNK), jnp.float32),
            pltpu.VMEM_SHARED((DEG_ROWS, CHUNK), jnp.float32),
        ],
    )
    def k(dst_hbm, z_hbm, out_hbm, dst_v, hist, acc):
        c = lax.axis_index("c")
        s = lax.axis_index("s")
        wid = c * NS + s
        rows_sub = 8
        r0 = s * rows_sub

        @pl.when(s < DEG_ROWS // rows_sub)
        def _():
            pltpu.sync_copy(z_hbm.at[pl.ds(r0, rows_sub)],
                            acc.at[pl.ds(r0, rows_sub)])

        pltpu.sync_copy(dst_hbm.at[wid], dst_v)

        def zero_row(j, carry):
            for l in range(CHUNK // 16):
                hist[j, pl.ds(16 * l, 16)] = jnp.zeros((16,), jnp.float32)
            return carry

        lax.fori_loop(0, DEG_ROWS, zero_row, 0)

        def body(j, carry):
            for l in range(CHUNK // 16):
                d = dst_v[j, pl.ds(16 * l, 16)]
                counts, last = plsc.scan_count(d)
                plsc.addupdate_scatter(
                    hist,
                    [lax.shift_right_logical(d, 7), jnp.bitwise_and(d, 127)],
                    counts.astype(jnp.float32), mask=last)
            return carry

        lax.fori_loop(0, k_chunks, body, 0)
        plsc.subcore_barrier()
        for i in range(DEG_ROWS // 16):
            idx = lax.iota(jnp.int32, 16) + 16 * i
            pltpu.sync_copy(hist.at[pl.ds(16 * i, 16)], acc.at[idx], add=True)
        plsc.subcore_barrier()

        @pl.when(s < DEG_ROWS // rows_sub)
        def _():
            pltpu.sync_copy(acc.at[pl.ds(r0, rows_sub)],
                            out_hbm.at[c, pl.ds(r0, rows_sub)])

    return k(dst_slab, zeros_deg)


def _sc_aggregate(table, ed_slab, zeros, k_chunks, feat):
    assert k_chunks % 4 == 0
    quads = k_chunks // 4

    @functools.partial(
        pl.kernel,
        out_type=jax.ShapeDtypeStruct((NC, N_NODES_PAD, feat), jnp.float32),
        mesh=_mesh,
        scratch_types=[
            pltpu.VMEM((2, 2, 2, CHUNK), jnp.int32),
            pltpu.VMEM((2, CHUNK, feat), jnp.float32),
            pltpu.VMEM_SHARED((N_NODES_PAD, feat), jnp.float32),
            pltpu.SemaphoreType.DMA,
            pltpu.SemaphoreType.DMA,
            pltpu.SemaphoreType.DMA,
        ],
    )
    def k(table_hbm, ed_hbm, z_hbm, out_hbm, ib, rows, acc, sem0, sem1, semi):
        c = lax.axis_index("c")
        s = lax.axis_index("s")
        wid = c * NS + s
        r0 = s * ROWS_PER_SUB
        pltpu.sync_copy(z_hbm.at[pl.ds(r0, ROWS_PER_SUB)],
                        acc.at[pl.ds(r0, ROWS_PER_SUB)])
        pltpu.sync_copy(ed_hbm.at[wid, pl.ds(0, 2)], ib.at[0])
        plsc.subcore_barrier()

        sems = (sem0, sem1)

        def g_start(b, p, cip):
            pltpu.async_copy(table_hbm.at[ib.at[p, cip, 0]], rows.at[b],
                             sems[b])

        def g_wait(b, p, cip):
            pltpu.make_async_copy(table_hbm.at[ib.at[p, cip, 0]], rows.at[b],
                                  sems[b]).wait()

        def s_start(b, p, cip):
            pltpu.async_copy(rows.at[b], acc.at[ib.at[p, cip, 1]], sems[b],
                             add=True)

        def s_wait(b, p, cip):
            pltpu.make_async_copy(rows.at[b], acc.at[ib.at[p, cip, 1]],
                                  sems[b]).wait()

        def i_start(j0, p):
            pltpu.async_copy(ed_hbm.at[wid, pl.ds(j0, 2)], ib.at[p], semi)

        def i_wait(j0, p):
            pltpu.make_async_copy(ed_hbm.at[wid, pl.ds(j0, 2)], ib.at[p],
                                  semi).wait()

        g_start(0, 0, 0)

        def body(u, carry):
            c0 = 4 * u

            @pl.when(u > 0)
            def _():
                s_wait(1, 1, 1)

            g_start(1, 0, 1)
            g_wait(0, 0, 0)
            s_start(0, 0, 0)
            i_start(c0 + 2, 1)
            g_wait(1, 0, 1)
            s_start(1, 0, 1)
            s_wait(0, 0, 0)
            i_wait(c0 + 2, 1)
            g_start(0, 1, 0)
            s_wait(1, 0, 1)
            g_start(1, 1, 1)
            g_wait(0, 1, 0)
            s_start(0, 1, 0)

            @pl.when(u + 1 < quads)
            def _():
                i_start(c0 + 4, 0)

            g_wait(1, 1, 1)
            s_start(1, 1, 1)
            s_wait(0, 1, 0)

            @pl.when(u + 1 < quads)
            def _():
                i_wait(c0 + 4, 0)
                g_start(0, 0, 0)

            return carry

        lax.fori_loop(0, quads, body, 0)
        s_wait(1, 1, 1)
        plsc.subcore_barrier()
        pltpu.sync_copy(acc.at[pl.ds(r0, ROWS_PER_SUB)],
                        out_hbm.at[c, pl.ds(r0, ROWS_PER_SUB)])

    return k(table, ed_slab, zeros)


def _dis_from(deg_ref):
    return lax.rsqrt(deg_ref[...] + 1.0)


def _tc_h1p(x, W1, degp):
    blk = 512
    hid = W1.shape[1]

    def body(x_ref, w_ref, d_ref, o_ref):
        dis = _dis_from(d_ref)
        h = jnp.dot(x_ref[...], w_ref[...], preferred_element_type=jnp.float32)
        o_ref[...] = h * dis

    return pl.pallas_call(
        body,
        grid=(N_NODES_PAD // blk,),
        in_specs=[
            pl.BlockSpec((blk, x.shape[1]), lambda i: (i, 0)),
            pl.BlockSpec((x.shape[1], hid), lambda i: (0, 0)),
            pl.BlockSpec((blk, 1), lambda i: (i, 0)),
        ],
        out_specs=pl.BlockSpec((blk, hid), lambda i: (i, 0)),
        out_shape=jax.ShapeDtypeStruct((N_NODES_PAD, hid), jnp.float32),
    )(x, W1, degp)


def _tc_hp2(a1, h1p, degp, b1):
    blk = 512
    hid = h1p.shape[1]

    def body(a_ref, h_ref, d_ref, b_ref, o_ref):
        dis = _dis_from(d_ref)
        tot = a_ref[0] + a_ref[1] + h_ref[...]
        o_ref[...] = jnp.maximum(tot * dis + b_ref[...], 0.0) * dis

    return pl.pallas_call(
        body,
        grid=(N_NODES_PAD // blk,),
        in_specs=[
            pl.BlockSpec((NC, blk, hid), lambda i: (0, i, 0)),
            pl.BlockSpec((blk, hid), lambda i: (i, 0)),
            pl.BlockSpec((blk, 1), lambda i: (i, 0)),
            pl.BlockSpec((1, hid), lambda i: (0, 0)),
        ],
        out_specs=pl.BlockSpec((blk, hid), lambda i: (i, 0)),
        out_shape=jax.ShapeDtypeStruct((N_NODES_PAD, hid), jnp.float32),
    )(a1, h1p, degp, b1.reshape(1, hid))


def _tc_final(a2, hp2, degp, W2, b2):
    blk = 512
    hid = hp2.shape[1]
    out_c = W2.shape[1]

    def body(a_ref, h_ref, d_ref, w_ref, b_ref, o_ref):
        dis = _dis_from(d_ref)
        tot = (a_ref[0] + a_ref[1] + h_ref[...]) * dis
        o_ref[...] = jnp.dot(tot, w_ref[...],
                             preferred_element_type=jnp.float32) + b_ref[...]

    return pl.pallas_call(
        body,
        grid=(N_NODES_PAD // blk,),
        in_specs=[
            pl.BlockSpec((NC, blk, hid), lambda i: (0, i, 0)),
            pl.BlockSpec((blk, hid), lambda i: (i, 0)),
            pl.BlockSpec((blk, 1), lambda i: (i, 0)),
            pl.BlockSpec((hid, out_c), lambda i: (0, 0)),
            pl.BlockSpec((1, out_c), lambda i: (0, 0)),
        ],
        out_specs=pl.BlockSpec((blk, out_c), lambda i: (i, 0)),
        out_shape=jax.ShapeDtypeStruct((N_NODES_PAD, out_c), jnp.float32),
    )(a2, hp2, degp, W2, b2.reshape(1, out_c))


def kernel(x, edge_index, W1, b1, W2, b2):
    n = x.shape[0]
    e = edge_index.shape[1]
    k_chunks = -(-e // (NW * CHUNK))
    k_chunks += (-k_chunks) % 4
    pad_e = NW * k_chunks * CHUNK

    src = edge_index[0].astype(jnp.int32)
    dst = edge_index[1].astype(jnp.int32)
    npad = pad_e - e
    pad_idx = n + (jnp.arange(npad, dtype=jnp.int32) % (N_NODES_PAD - n))
    src_slab = jnp.concatenate([src, pad_idx]).reshape(NW, k_chunks, CHUNK)
    dst_slab = jnp.concatenate([dst, pad_idx]).reshape(NW, k_chunks, CHUNK)
    ed_slab = jnp.stack([src_slab, dst_slab], axis=2)

    xp = jnp.zeros((N_NODES_PAD, x.shape[1]), jnp.float32).at[:n].set(x)
    z_deg = jnp.zeros((DEG_ROWS, CHUNK), jnp.float32)
    z_hid = jnp.zeros((N_NODES_PAD, W1.shape[1]), jnp.float32)

    degp = _sc_degree(dst_slab, z_deg, k_chunks)
    deg = (degp[0] + degp[1]).reshape(N_NODES_PAD, 1)
    h1p = _tc_h1p(xp, W1, deg)
    a1 = _sc_aggregate(h1p, ed_slab, z_hid, k_chunks, W1.shape[1])
    hp2 = _tc_hp2(a1, h1p, deg, b1)
    a2 = _sc_aggregate(hp2, ed_slab, z_hid, k_chunks, W1.shape[1])
    out = _tc_final(a2, hp2, deg, W2, b2)
    return out[:n]


# speedup vs baseline: 1.0676x; 1.0676x over previous
import functools

import jax
import jax.numpy as jnp
from jax import lax
from jax.experimental import pallas as pl
from jax.experimental.pallas import tpu as pltpu
from jax.experimental.pallas import tpu_sc as plsc

N_NODES_PAD = 10240
NC = 2
NS = 16
NW = NC * NS
CHUNK = 128
ROWS_PER_SUB = N_NODES_PAD // NS

_mesh = plsc.VectorSubcoreMesh(core_axis_name="c", subcore_axis_name="s")


DEG_ROWS = N_NODES_PAD // CHUNK


def _sc_degree(dst_slab, zeros_deg, k_chunks):
    @functools.partial(
        pl.kernel,
        out_type=jax.ShapeDtypeStruct((NC, DEG_ROWS, CHUNK), jnp.float32),
        mesh=_mesh,
        compiler_params=pltpu.CompilerParams(needs_layout_passes=False),
        scratch_types=[
            pltpu.VMEM((k_chunks, CHUNK), jnp.int32),
            pltpu.VMEM((DEG_ROWS, CHUNK), jnp.float32),
            pltpu.VMEM_SHARED((DEG_ROWS, CHUNK), jnp.float32),
        ],
    )
    def k(dst_hbm, z_hbm, out_hbm, dst_v, hist, acc):
        c = lax.axis_index("c")
        s = lax.axis_index("s")
        wid = c * NS + s
        rows_sub = 8
        r0 = s * rows_sub

        @pl.when(s < DEG_ROWS // rows_sub)
        def _():
            pltpu.sync_copy(z_hbm.at[pl.ds(r0, rows_sub)],
                            acc.at[pl.ds(r0, rows_sub)])

        pltpu.sync_copy(dst_hbm.at[wid], dst_v)

        def zero_row(j, carry):
            for l in range(CHUNK // 16):
                hist[j, pl.ds(16 * l, 16)] = jnp.zeros((16,), jnp.float32)
            return carry

        lax.fori_loop(0, DEG_ROWS, zero_row, 0)

        def body(j, carry):
            for l in range(CHUNK // 16):
                d = dst_v[j, pl.ds(16 * l, 16)]
                counts, last = plsc.scan_count(d)
                plsc.addupdate_scatter(
                    hist,
                    [lax.shift_right_logical(d, 7), jnp.bitwise_and(d, 127)],
                    counts.astype(jnp.float32), mask=last)
            return carry

        lax.fori_loop(0, k_chunks, body, 0)
        plsc.subcore_barrier()
        for i in range(DEG_ROWS // 16):
            idx = lax.iota(jnp.int32, 16) + 16 * i
            pltpu.sync_copy(hist.at[pl.ds(16 * i, 16)], acc.at[idx], add=True)
        plsc.subcore_barrier()

        @pl.when(s < DEG_ROWS // rows_sub)
        def _():
            pltpu.sync_copy(acc.at[pl.ds(r0, rows_sub)],
                            out_hbm.at[c, pl.ds(r0, rows_sub)])

    return k(dst_slab, zeros_deg)


def _sc_aggregate(table, ed_slab, zeros, k_chunks, feat):
    assert k_chunks % 4 == 0
    quads = k_chunks // 4

    @functools.partial(
        pl.kernel,
        out_type=jax.ShapeDtypeStruct((NC, N_NODES_PAD, feat), jnp.float32),
        mesh=_mesh,
        scratch_types=[
            pltpu.VMEM((2, 2, 2, CHUNK), jnp.int32),
            pltpu.VMEM((2, CHUNK, feat), jnp.float32),
            pltpu.VMEM_SHARED((N_NODES_PAD, feat), jnp.float32),
            pltpu.SemaphoreType.DMA,
            pltpu.SemaphoreType.DMA,
            pltpu.SemaphoreType.DMA,
        ],
    )
    def k(table_hbm, ed_hbm, z_hbm, out_hbm, ib, rows, acc, sem0, sem1, semi):
        c = lax.axis_index("c")
        s = lax.axis_index("s")
        wid = c * NS + s
        r0 = s * ROWS_PER_SUB
        pltpu.sync_copy(z_hbm.at[pl.ds(r0, ROWS_PER_SUB)],
                        acc.at[pl.ds(r0, ROWS_PER_SUB)])
        pltpu.sync_copy(ed_hbm.at[wid, pl.ds(0, 2)], ib.at[0])
        plsc.subcore_barrier()

        sems = (sem0, sem1)

        def g_start(b, p, cip):
            pltpu.async_copy(table_hbm.at[ib.at[p, cip, 0]], rows.at[b],
                             sems[b])

        def g_wait(b, p, cip):
            pltpu.make_async_copy(table_hbm.at[ib.at[p, cip, 0]], rows.at[b],
                                  sems[b]).wait()

        def s_start(b, p, cip):
            pltpu.async_copy(rows.at[b], acc.at[ib.at[p, cip, 1]], sems[b],
                             add=True)

        def s_wait(b, p, cip):
            pltpu.make_async_copy(rows.at[b], acc.at[ib.at[p, cip, 1]],
                                  sems[b]).wait()

        def i_start(j0, p):
            pltpu.async_copy(ed_hbm.at[wid, pl.ds(j0, 2)], ib.at[p], semi)

        def i_wait(j0, p):
            pltpu.make_async_copy(ed_hbm.at[wid, pl.ds(j0, 2)], ib.at[p],
                                  semi).wait()

        g_start(0, 0, 0)

        def body(u, carry):
            c0 = 4 * u
            g_wait(0, 0, 0)
            s_start(0, 0, 0)

            @pl.when(u > 0)
            def _():
                s_wait(1, 1, 1)

            i_start(c0 + 2, 1)
            g_start(1, 0, 1)
            g_wait(1, 0, 1)
            s_start(1, 0, 1)
            s_wait(0, 0, 0)
            i_wait(c0 + 2, 1)
            g_start(0, 1, 0)
            g_wait(0, 1, 0)
            s_start(0, 1, 0)
            s_wait(1, 0, 1)

            @pl.when(u + 1 < quads)
            def _():
                i_start(c0 + 4, 0)

            g_start(1, 1, 1)
            g_wait(1, 1, 1)
            s_start(1, 1, 1)
            s_wait(0, 1, 0)

            @pl.when(u + 1 < quads)
            def _():
                i_wait(c0 + 4, 0)
                g_start(0, 0, 0)

            return carry

        lax.fori_loop(0, quads, body, 0)
        s_wait(1, 1, 1)
        plsc.subcore_barrier()
        pltpu.sync_copy(acc.at[pl.ds(r0, ROWS_PER_SUB)],
                        out_hbm.at[c, pl.ds(r0, ROWS_PER_SUB)])

    return k(table, ed_slab, zeros)


def _dis_from(deg_ref):
    return lax.rsqrt(deg_ref[...] + 1.0)


def _tc_h1(x, W1):
    blk = 512
    hid = W1.shape[1]

    def body(x_ref, w_ref, o_ref):
        o_ref[...] = jnp.dot(x_ref[...], w_ref[...],
                             preferred_element_type=jnp.float32)

    return pl.pallas_call(
        body,
        grid=(N_NODES_PAD // blk,),
        in_specs=[
            pl.BlockSpec((blk, x.shape[1]), lambda i: (i, 0)),
            pl.BlockSpec((x.shape[1], hid), lambda i: (0, 0)),
        ],
        out_specs=pl.BlockSpec((blk, hid), lambda i: (i, 0)),
        out_shape=jax.ShapeDtypeStruct((N_NODES_PAD, hid), jnp.float32),
    )(x, W1)


def _tc_scale(h1, degp):
    blk = 512
    hid = h1.shape[1]

    def body(h_ref, d_ref, o_ref):
        o_ref[...] = h_ref[...] * _dis_from(d_ref)

    return pl.pallas_call(
        body,
        grid=(N_NODES_PAD // blk,),
        in_specs=[
            pl.BlockSpec((blk, hid), lambda i: (i, 0)),
            pl.BlockSpec((blk, 1), lambda i: (i, 0)),
        ],
        out_specs=pl.BlockSpec((blk, hid), lambda i: (i, 0)),
        out_shape=jax.ShapeDtypeStruct((N_NODES_PAD, hid), jnp.float32),
    )(h1, degp)


def _tc_hp2(a1, h1p, degp, b1):
    blk = 512
    hid = h1p.shape[1]

    def body(a_ref, h_ref, d_ref, b_ref, o_ref):
        dis = _dis_from(d_ref)
        tot = a_ref[0] + a_ref[1] + h_ref[...]
        o_ref[...] = jnp.maximum(tot * dis + b_ref[...], 0.0) * dis

    return pl.pallas_call(
        body,
        grid=(N_NODES_PAD // blk,),
        in_specs=[
            pl.BlockSpec((NC, blk, hid), lambda i: (0, i, 0)),
            pl.BlockSpec((blk, hid), lambda i: (i, 0)),
            pl.BlockSpec((blk, 1), lambda i: (i, 0)),
            pl.BlockSpec((1, hid), lambda i: (0, 0)),
        ],
        out_specs=pl.BlockSpec((blk, hid), lambda i: (i, 0)),
        out_shape=jax.ShapeDtypeStruct((N_NODES_PAD, hid), jnp.float32),
    )(a1, h1p, degp, b1.reshape(1, hid))


def _tc_final(a2, hp2, degp, W2, b2):
    blk = 512
    hid = hp2.shape[1]
    out_c = W2.shape[1]

    def body(a_ref, h_ref, d_ref, w_ref, b_ref, o_ref):
        dis = _dis_from(d_ref)
        tot = (a_ref[0] + a_ref[1] + h_ref[...]) * dis
        o_ref[...] = jnp.dot(tot, w_ref[...],
                             preferred_element_type=jnp.float32) + b_ref[...]

    return pl.pallas_call(
        body,
        grid=(N_NODES_PAD // blk,),
        in_specs=[
            pl.BlockSpec((NC, blk, hid), lambda i: (0, i, 0)),
            pl.BlockSpec((blk, hid), lambda i: (i, 0)),
            pl.BlockSpec((blk, 1), lambda i: (i, 0)),
            pl.BlockSpec((hid, out_c), lambda i: (0, 0)),
            pl.BlockSpec((1, out_c), lambda i: (0, 0)),
        ],
        out_specs=pl.BlockSpec((blk, out_c), lambda i: (i, 0)),
        out_shape=jax.ShapeDtypeStruct((N_NODES_PAD, out_c), jnp.float32),
    )(a2, hp2, degp, W2, b2.reshape(1, out_c))


def kernel(x, edge_index, W1, b1, W2, b2):
    n = x.shape[0]
    e = edge_index.shape[1]
    k_chunks = -(-e // (NW * CHUNK))
    k_chunks += (-k_chunks) % 4
    pad_e = NW * k_chunks * CHUNK

    src = edge_index[0].astype(jnp.int32)
    dst = edge_index[1].astype(jnp.int32)
    npad = pad_e - e
    pad_idx = n + (jnp.arange(npad, dtype=jnp.int32) % (N_NODES_PAD - n))
    src_slab = jnp.concatenate([src, pad_idx]).reshape(NW, k_chunks, CHUNK)
    dst_slab = jnp.concatenate([dst, pad_idx]).reshape(NW, k_chunks, CHUNK)
    ed_slab = jnp.stack([src_slab, dst_slab], axis=2)

    xp = jnp.zeros((N_NODES_PAD, x.shape[1]), jnp.float32).at[:n].set(x)
    z_deg = jnp.zeros((DEG_ROWS, CHUNK), jnp.float32)
    z_hid = jnp.zeros((N_NODES_PAD, W1.shape[1]), jnp.float32)

    degp = _sc_degree(dst_slab, z_deg, k_chunks)
    h1 = _tc_h1(xp, W1)
    deg = (degp[0] + degp[1]).reshape(N_NODES_PAD, 1)
    h1p = _tc_scale(h1, deg)
    a1 = _sc_aggregate(h1p, ed_slab, z_hid, k_chunks, W1.shape[1])
    hp2 = _tc_hp2(a1, h1p, deg, b1)
    a2 = _sc_aggregate(hp2, ed_slab, z_hid, k_chunks, W1.shape[1])
    out = _tc_final(a2, hp2, deg, W2, b2)
    return out[:n]


# speedup vs baseline: 1.1002x; 1.0305x over previous
import functools

import jax
import jax.numpy as jnp
from jax import lax
from jax.experimental import pallas as pl
from jax.experimental.pallas import tpu as pltpu
from jax.experimental.pallas import tpu_sc as plsc

N_NODES_PAD = 10240
NC = 2
NS = 16
NW = NC * NS
CHUNK = 128
ROWS_PER_SUB = N_NODES_PAD // NS

_mesh = plsc.VectorSubcoreMesh(core_axis_name="c", subcore_axis_name="s")


DEG_ROWS = N_NODES_PAD // CHUNK


def _sc_degree(dst_slab, zeros_deg, k_chunks):
    @functools.partial(
        pl.kernel,
        out_type=jax.ShapeDtypeStruct((NC, DEG_ROWS, CHUNK), jnp.float32),
        mesh=_mesh,
        compiler_params=pltpu.CompilerParams(needs_layout_passes=False),
        scratch_types=[
            pltpu.VMEM((k_chunks, CHUNK), jnp.int32),
            pltpu.VMEM((DEG_ROWS, CHUNK), jnp.float32),
            pltpu.VMEM_SHARED((DEG_ROWS, CHUNK), jnp.float32),
        ],
    )
    def k(dst_hbm, z_hbm, out_hbm, dst_v, hist, acc):
        c = lax.axis_index("c")
        s = lax.axis_index("s")
        wid = c * NS + s
        rows_sub = 8
        r0 = s * rows_sub

        @pl.when(s < DEG_ROWS // rows_sub)
        def _():
            pltpu.sync_copy(z_hbm.at[pl.ds(r0, rows_sub)],
                            acc.at[pl.ds(r0, rows_sub)])

        pltpu.sync_copy(dst_hbm.at[wid], dst_v)

        def zero_row(j, carry):
            for l in range(CHUNK // 16):
                hist[j, pl.ds(16 * l, 16)] = jnp.zeros((16,), jnp.float32)
            return carry

        lax.fori_loop(0, DEG_ROWS, zero_row, 0)

        def body(j, carry):
            for l in range(CHUNK // 16):
                d = dst_v[j, pl.ds(16 * l, 16)]
                counts, last = plsc.scan_count(d)
                plsc.addupdate_scatter(
                    hist,
                    [lax.shift_right_logical(d, 7), jnp.bitwise_and(d, 127)],
                    counts.astype(jnp.float32), mask=last)
            return carry

        lax.fori_loop(0, k_chunks, body, 0)
        plsc.subcore_barrier()
        for i in range(DEG_ROWS // 16):
            idx = lax.iota(jnp.int32, 16) + 16 * i
            pltpu.sync_copy(hist.at[pl.ds(16 * i, 16)], acc.at[idx], add=True)
        plsc.subcore_barrier()

        @pl.when(s < DEG_ROWS // rows_sub)
        def _():
            pltpu.sync_copy(acc.at[pl.ds(r0, rows_sub)],
                            out_hbm.at[c, pl.ds(r0, rows_sub)])

    return k(dst_slab, zeros_deg)


def _sc_aggregate(table, ed_slab, k_chunks, feat):
    assert k_chunks % 4 == 0
    quads = k_chunks // 4

    @functools.partial(
        pl.kernel,
        out_type=jax.ShapeDtypeStruct((NC, N_NODES_PAD, feat), jnp.float32),
        mesh=_mesh,
        scratch_types=[
            pltpu.VMEM((2, 2, 2, CHUNK), jnp.int32),
            pltpu.VMEM((2, CHUNK, feat), jnp.float32),
            pltpu.VMEM_SHARED((N_NODES_PAD, feat), jnp.float32),
            pltpu.SemaphoreType.DMA,
            pltpu.SemaphoreType.DMA,
            pltpu.SemaphoreType.DMA,
        ],
    )
    def k(table_hbm, ed_hbm, out_hbm, ib, rows, acc, sem0, sem1, semi):
        c = lax.axis_index("c")
        s = lax.axis_index("s")
        wid = c * NS + s
        r0 = s * ROWS_PER_SUB
        pltpu.sync_copy(table_hbm.at[pl.ds(r0, ROWS_PER_SUB)],
                        acc.at[pl.ds(r0, ROWS_PER_SUB)])
        pltpu.sync_copy(ed_hbm.at[wid, pl.ds(0, 2)], ib.at[0])
        plsc.subcore_barrier()

        sems = (sem0, sem1)

        def g_start(b, p, cip):
            pltpu.async_copy(table_hbm.at[ib.at[p, cip, 0]], rows.at[b],
                             sems[b])

        def g_wait(b, p, cip):
            pltpu.make_async_copy(table_hbm.at[ib.at[p, cip, 0]], rows.at[b],
                                  sems[b]).wait()

        def s_start(b, p, cip):
            pltpu.async_copy(rows.at[b], acc.at[ib.at[p, cip, 1]], sems[b],
                             add=True)

        def s_wait(b, p, cip):
            pltpu.make_async_copy(rows.at[b], acc.at[ib.at[p, cip, 1]],
                                  sems[b]).wait()

        def i_start(j0, p):
            pltpu.async_copy(ed_hbm.at[wid, pl.ds(j0, 2)], ib.at[p], semi)

        def i_wait(j0, p):
            pltpu.make_async_copy(ed_hbm.at[wid, pl.ds(j0, 2)], ib.at[p],
                                  semi).wait()

        g_start(0, 0, 0)

        def body(u, carry):
            c0 = 4 * u
            g_wait(0, 0, 0)
            s_start(0, 0, 0)

            @pl.when(u > 0)
            def _():
                s_wait(1, 1, 1)

            i_start(c0 + 2, 1)
            g_start(1, 0, 1)
            g_wait(1, 0, 1)
            s_start(1, 0, 1)
            s_wait(0, 0, 0)
            i_wait(c0 + 2, 1)
            g_start(0, 1, 0)
            g_wait(0, 1, 0)
            s_start(0, 1, 0)
            s_wait(1, 0, 1)

            @pl.when(u + 1 < quads)
            def _():
                i_start(c0 + 4, 0)

            g_start(1, 1, 1)
            g_wait(1, 1, 1)
            s_start(1, 1, 1)
            s_wait(0, 1, 0)

            @pl.when(u + 1 < quads)
            def _():
                i_wait(c0 + 4, 0)
                g_start(0, 0, 0)

            return carry

        lax.fori_loop(0, quads, body, 0)
        s_wait(1, 1, 1)
        plsc.subcore_barrier()
        pltpu.sync_copy(acc.at[pl.ds(r0, ROWS_PER_SUB)],
                        out_hbm.at[c, pl.ds(r0, ROWS_PER_SUB)])

    return k(table, ed_slab)


def _dis_from(deg_ref):
    return lax.rsqrt(deg_ref[...] + 1.0)


def _tc_h1p(x, W1, degp):
    blk = 512
    hid = W1.shape[1]

    def body(x_ref, w_ref, d_ref, o_ref):
        dis = _dis_from(d_ref)
        h = jnp.dot(x_ref[...], w_ref[...], preferred_element_type=jnp.float32)
        o_ref[...] = h * dis

    return pl.pallas_call(
        body,
        grid=(N_NODES_PAD // blk,),
        in_specs=[
            pl.BlockSpec((blk, x.shape[1]), lambda i: (i, 0)),
            pl.BlockSpec((x.shape[1], hid), lambda i: (0, 0)),
            pl.BlockSpec((blk, 1), lambda i: (i, 0)),
        ],
        out_specs=pl.BlockSpec((blk, hid), lambda i: (i, 0)),
        out_shape=jax.ShapeDtypeStruct((N_NODES_PAD, hid), jnp.float32),
    )(x, W1, degp)


def _tc_hp2(a1, h1p, degp, b1):
    blk = 512
    hid = h1p.shape[1]

    def body(a_ref, h_ref, d_ref, b_ref, o_ref):
        dis = _dis_from(d_ref)
        tot = a_ref[0] + a_ref[1] - h_ref[...]
        o_ref[...] = jnp.maximum(tot * dis + b_ref[...], 0.0) * dis

    return pl.pallas_call(
        body,
        grid=(N_NODES_PAD // blk,),
        in_specs=[
            pl.BlockSpec((NC, blk, hid), lambda i: (0, i, 0)),
            pl.BlockSpec((blk, hid), lambda i: (i, 0)),
            pl.BlockSpec((blk, 1), lambda i: (i, 0)),
            pl.BlockSpec((1, hid), lambda i: (0, 0)),
        ],
        out_specs=pl.BlockSpec((blk, hid), lambda i: (i, 0)),
        out_shape=jax.ShapeDtypeStruct((N_NODES_PAD, hid), jnp.float32),
    )(a1, h1p, degp, b1.reshape(1, hid))


def _tc_final(a2, hp2, degp, W2, b2, n):
    blk = 512
    hid = hp2.shape[1]
    out_c = W2.shape[1]

    def body(a_ref, h_ref, d_ref, w_ref, b_ref, o_ref):
        dis = _dis_from(d_ref)
        tot = (a_ref[0] + a_ref[1] - h_ref[...]) * dis
        o_ref[...] = jnp.dot(tot, w_ref[...],
                             preferred_element_type=jnp.float32) + b_ref[...]

    return pl.pallas_call(
        body,
        grid=(N_NODES_PAD // blk,),
        in_specs=[
            pl.BlockSpec((NC, blk, hid), lambda i: (0, i, 0)),
            pl.BlockSpec((blk, hid), lambda i: (i, 0)),
            pl.BlockSpec((blk, 1), lambda i: (i, 0)),
            pl.BlockSpec((hid, out_c), lambda i: (0, 0)),
            pl.BlockSpec((1, out_c), lambda i: (0, 0)),
        ],
        out_specs=pl.BlockSpec((blk, out_c), lambda i: (i, 0)),
        out_shape=jax.ShapeDtypeStruct((n, out_c), jnp.float32),
    )(a2, hp2, degp, W2, b2.reshape(1, out_c))


def kernel(x, edge_index, W1, b1, W2, b2):
    n = x.shape[0]
    e = edge_index.shape[1]
    k_chunks = -(-e // (NW * CHUNK))
    k_chunks += (-k_chunks) % 4
    pad_e = NW * k_chunks * CHUNK

    src = edge_index[0].astype(jnp.int32)
    dst = edge_index[1].astype(jnp.int32)
    npad = pad_e - e
    pad_idx = n + (jnp.arange(npad, dtype=jnp.int32) % (N_NODES_PAD - n))
    src_slab = jnp.concatenate([src, pad_idx]).reshape(NW, k_chunks, CHUNK)
    dst_slab = jnp.concatenate([dst, pad_idx]).reshape(NW, k_chunks, CHUNK)
    ed_slab = jnp.stack([src_slab, dst_slab], axis=2)

    z_deg = jnp.zeros((DEG_ROWS, CHUNK), jnp.float32)

    degp = _sc_degree(dst_slab, z_deg, k_chunks)
    deg = (degp[0] + degp[1]).reshape(N_NODES_PAD, 1)
    h1p = _tc_h1p(x, W1, deg)
    a1 = _sc_aggregate(h1p, ed_slab, k_chunks, W1.shape[1])
    hp2 = _tc_hp2(a1, h1p, deg, b1)
    a2 = _sc_aggregate(hp2, ed_slab, k_chunks, W1.shape[1])
    return _tc_final(a2, hp2, deg, W2, b2, n)


# speedup vs baseline: 1.1131x; 1.0118x over previous
import functools

import jax
import jax.numpy as jnp
from jax import lax
from jax.experimental import pallas as pl
from jax.experimental.pallas import tpu as pltpu
from jax.experimental.pallas import tpu_sc as plsc

N_NODES_PAD = 10240
NC = 2
NS = 16
NW = NC * NS
CHUNK = 128
ROWS_PER_SUB = N_NODES_PAD // NS

_mesh = plsc.VectorSubcoreMesh(core_axis_name="c", subcore_axis_name="s")


DEG_ROWS = N_NODES_PAD // CHUNK


def _sc_degree(dst_slab, zeros_deg, k_chunks):
    @functools.partial(
        pl.kernel,
        out_type=jax.ShapeDtypeStruct((NC, DEG_ROWS, CHUNK), jnp.float32),
        mesh=_mesh,
        compiler_params=pltpu.CompilerParams(needs_layout_passes=False),
        scratch_types=[
            pltpu.VMEM((k_chunks, CHUNK), jnp.int32),
            pltpu.VMEM((DEG_ROWS, CHUNK), jnp.float32),
            pltpu.VMEM_SHARED((DEG_ROWS, CHUNK), jnp.float32),
        ],
    )
    def k(dst_hbm, z_hbm, out_hbm, dst_v, hist, acc):
        c = lax.axis_index("c")
        s = lax.axis_index("s")
        wid = c * NS + s
        rows_sub = 8
        r0 = s * rows_sub

        @pl.when(s < DEG_ROWS // rows_sub)
        def _():
            pltpu.sync_copy(z_hbm.at[pl.ds(r0, rows_sub)],
                            acc.at[pl.ds(r0, rows_sub)])

        pltpu.sync_copy(dst_hbm.at[wid], dst_v)

        def zero_row(j, carry):
            for l in range(CHUNK // 16):
                hist[j, pl.ds(16 * l, 16)] = jnp.zeros((16,), jnp.float32)
            return carry

        lax.fori_loop(0, DEG_ROWS, zero_row, 0)

        ones = jnp.full((16,), 1.0, jnp.float32)

        def body(j, carry):
            for l in range(CHUNK // 16):
                d = dst_v[j, pl.ds(16 * l, 16)]
                plsc.addupdate_scatter(
                    hist,
                    [lax.shift_right_logical(d, 7), jnp.bitwise_and(d, 127)],
                    ones)
            return carry

        lax.fori_loop(0, k_chunks, body, 0)
        plsc.subcore_barrier()
        for i in range(DEG_ROWS // 16):
            idx = lax.iota(jnp.int32, 16) + 16 * i
            pltpu.sync_copy(hist.at[pl.ds(16 * i, 16)], acc.at[idx], add=True)
        plsc.subcore_barrier()

        @pl.when(s < DEG_ROWS // rows_sub)
        def _():
            pltpu.sync_copy(acc.at[pl.ds(r0, rows_sub)],
                            out_hbm.at[c, pl.ds(r0, rows_sub)])

    return k(dst_slab, zeros_deg)


def _sc_aggregate(table, ed_slab, k_chunks, feat):
    assert k_chunks % 4 == 0
    quads = k_chunks // 4

    @functools.partial(
        pl.kernel,
        out_type=jax.ShapeDtypeStruct((NC, N_NODES_PAD, feat), jnp.float32),
        mesh=_mesh,
        scratch_types=[
            pltpu.VMEM((2, 2, 2, CHUNK), jnp.int32),
            pltpu.VMEM((2, CHUNK, feat), jnp.float32),
            pltpu.VMEM_SHARED((N_NODES_PAD, feat), jnp.float32),
            pltpu.SemaphoreType.DMA,
            pltpu.SemaphoreType.DMA,
            pltpu.SemaphoreType.DMA,
        ],
    )
    def k(table_hbm, ed_hbm, out_hbm, ib, rows, acc, sem0, sem1, semi):
        c = lax.axis_index("c")
        s = lax.axis_index("s")
        wid = c * NS + s
        r0 = s * ROWS_PER_SUB
        pltpu.sync_copy(table_hbm.at[pl.ds(r0, ROWS_PER_SUB)],
                        acc.at[pl.ds(r0, ROWS_PER_SUB)])
        pltpu.sync_copy(ed_hbm.at[wid, pl.ds(0, 2)], ib.at[0])
        plsc.subcore_barrier()

        sems = (sem0, sem1)

        def g_start(b, p, cip):
            pltpu.async_copy(table_hbm.at[ib.at[p, cip, 0]], rows.at[b],
                             sems[b])

        def g_wait(b, p, cip):
            pltpu.make_async_copy(table_hbm.at[ib.at[p, cip, 0]], rows.at[b],
                                  sems[b]).wait()

        def s_start(b, p, cip):
            pltpu.async_copy(rows.at[b], acc.at[ib.at[p, cip, 1]], sems[b],
                             add=True)

        def s_wait(b, p, cip):
            pltpu.make_async_copy(rows.at[b], acc.at[ib.at[p, cip, 1]],
                                  sems[b]).wait()

        def i_start(j0, p):
            pltpu.async_copy(ed_hbm.at[wid, pl.ds(j0, 2)], ib.at[p], semi)

        def i_wait(j0, p):
            pltpu.make_async_copy(ed_hbm.at[wid, pl.ds(j0, 2)], ib.at[p],
                                  semi).wait()

        g_start(0, 0, 0)

        def body(u, carry):
            c0 = 4 * u
            g_wait(0, 0, 0)
            s_start(0, 0, 0)

            @pl.when(u > 0)
            def _():
                s_wait(1, 1, 1)

            i_start(c0 + 2, 1)
            g_start(1, 0, 1)
            g_wait(1, 0, 1)
            s_start(1, 0, 1)
            s_wait(0, 0, 0)
            i_wait(c0 + 2, 1)
            g_start(0, 1, 0)
            g_wait(0, 1, 0)
            s_start(0, 1, 0)
            s_wait(1, 0, 1)

            @pl.when(u + 1 < quads)
            def _():
                i_start(c0 + 4, 0)

            g_start(1, 1, 1)
            g_wait(1, 1, 1)
            s_start(1, 1, 1)
            s_wait(0, 1, 0)

            @pl.when(u + 1 < quads)
            def _():
                i_wait(c0 + 4, 0)
                g_start(0, 0, 0)

            return carry

        lax.fori_loop(0, quads, body, 0)
        s_wait(1, 1, 1)
        plsc.subcore_barrier()
        pltpu.sync_copy(acc.at[pl.ds(r0, ROWS_PER_SUB)],
                        out_hbm.at[c, pl.ds(r0, ROWS_PER_SUB)])

    return k(table, ed_slab)


def _dis_from(deg_ref):
    return lax.rsqrt(deg_ref[...] + 1.0)


def _tc_h1p(x, W1, degp):
    blk = 512
    hid = W1.shape[1]

    def body(x_ref, w_ref, d_ref, o_ref):
        dis = _dis_from(d_ref)
        h = jnp.dot(x_ref[...], w_ref[...], preferred_element_type=jnp.float32)
        o_ref[...] = h * dis

    return pl.pallas_call(
        body,
        grid=(N_NODES_PAD // blk,),
        in_specs=[
            pl.BlockSpec((blk, x.shape[1]), lambda i: (i, 0)),
            pl.BlockSpec((x.shape[1], hid), lambda i: (0, 0)),
            pl.BlockSpec((blk, 1), lambda i: (i, 0)),
        ],
        out_specs=pl.BlockSpec((blk, hid), lambda i: (i, 0)),
        out_shape=jax.ShapeDtypeStruct((N_NODES_PAD, hid), jnp.float32),
    )(x, W1, degp)


def _tc_hp2(a1, h1p, degp, b1):
    blk = 512
    hid = h1p.shape[1]

    def body(a_ref, h_ref, d_ref, b_ref, o_ref):
        dis = _dis_from(d_ref)
        tot = a_ref[0] + a_ref[1] - h_ref[...]
        o_ref[...] = jnp.maximum(tot * dis + b_ref[...], 0.0) * dis

    return pl.pallas_call(
        body,
        grid=(N_NODES_PAD // blk,),
        in_specs=[
            pl.BlockSpec((NC, blk, hid), lambda i: (0, i, 0)),
            pl.BlockSpec((blk, hid), lambda i: (i, 0)),
            pl.BlockSpec((blk, 1), lambda i: (i, 0)),
            pl.BlockSpec((1, hid), lambda i: (0, 0)),
        ],
        out_specs=pl.BlockSpec((blk, hid), lambda i: (i, 0)),
        out_shape=jax.ShapeDtypeStruct((N_NODES_PAD, hid), jnp.float32),
    )(a1, h1p, degp, b1.reshape(1, hid))


def _tc_final(a2, hp2, degp, W2, b2, n):
    blk = 512
    hid = hp2.shape[1]
    out_c = W2.shape[1]

    def body(a_ref, h_ref, d_ref, w_ref, b_ref, o_ref):
        dis = _dis_from(d_ref)
        tot = (a_ref[0] + a_ref[1] - h_ref[...]) * dis
        o_ref[...] = jnp.dot(tot, w_ref[...],
                             preferred_element_type=jnp.float32) + b_ref[...]

    return pl.pallas_call(
        body,
        grid=(N_NODES_PAD // blk,),
        in_specs=[
            pl.BlockSpec((NC, blk, hid), lambda i: (0, i, 0)),
            pl.BlockSpec((blk, hid), lambda i: (i, 0)),
            pl.BlockSpec((blk, 1), lambda i: (i, 0)),
            pl.BlockSpec((hid, out_c), lambda i: (0, 0)),
            pl.BlockSpec((1, out_c), lambda i: (0, 0)),
        ],
        out_specs=pl.BlockSpec((blk, out_c), lambda i: (i, 0)),
        out_shape=jax.ShapeDtypeStruct((n, out_c), jnp.float32),
    )(a2, hp2, degp, W2, b2.reshape(1, out_c))


def kernel(x, edge_index, W1, b1, W2, b2):
    n = x.shape[0]
    e = edge_index.shape[1]
    k_chunks = -(-e // (NW * CHUNK))
    k_chunks += (-k_chunks) % 4
    pad_e = NW * k_chunks * CHUNK

    src = edge_index[0].astype(jnp.int32)
    dst = edge_index[1].astype(jnp.int32)
    npad = pad_e - e
    pad_idx = n + (jnp.arange(npad, dtype=jnp.int32) % (N_NODES_PAD - n))
    src_slab = jnp.concatenate([src, pad_idx]).reshape(NW, k_chunks, CHUNK)
    dst_slab = jnp.concatenate([dst, pad_idx]).reshape(NW, k_chunks, CHUNK)
    ed_slab = jnp.stack([src_slab, dst_slab], axis=2)

    z_deg = jnp.zeros((DEG_ROWS, CHUNK), jnp.float32)

    degp = _sc_degree(dst_slab, z_deg, k_chunks)
    deg = (degp[0] + degp[1]).reshape(N_NODES_PAD, 1)
    h1p = _tc_h1p(x, W1, deg)
    a1 = _sc_aggregate(h1p, ed_slab, k_chunks, W1.shape[1])
    hp2 = _tc_hp2(a1, h1p, deg, b1)
    a2 = _sc_aggregate(hp2, ed_slab, k_chunks, W1.shape[1])
    return _tc_final(a2, hp2, deg, W2, b2, n)
